# baseline (speedup 1.0000x reference)
import functools

import jax
import jax.numpy as jnp
from jax import lax
from jax.experimental import pallas as pl
from jax.experimental.pallas import tpu as pltpu
from jax.experimental.pallas import tpu_sc as plsc

N = 10000
D = 128
E = 320000
DIAG_LAMBDA = 0.2

NC = 2
NS = 16
NW = NC * NS
EPW = E // NW
CHUNK = 80
NREAL = EPW // CHUNK
NCHUNK = 126
EPAD = NCHUNK * CHUNK
N_PAD = 10240
RPT = N_PAD // NS


def _sc_aggregate(x, row, col):
    mesh = plsc.VectorSubcoreMesh(core_axis_name="c", subcore_axis_name="s")

    @functools.partial(
        pl.kernel,
        out_type=(
            jax.ShapeDtypeStruct((NC * N_PAD, D), jnp.float32),
            jax.ShapeDtypeStruct((NC * N_PAD,), jnp.float32),
        ),
        mesh=mesh,
        scratch_types=[
            pltpu.VMEM((EPAD,), jnp.int32),
            pltpu.VMEM((NCHUNK, CHUNK), jnp.int32),
            pltpu.VMEM((CHUNK, D), jnp.float32),
            pltpu.VMEM((CHUNK, D), jnp.float32),
            pltpu.VMEM((RPT,), jnp.float32),
            pltpu.VMEM((CHUNK,), jnp.float32),
            pltpu.VMEM_SHARED((N_PAD, D), jnp.float32),
            pltpu.VMEM_SHARED((N_PAD,), jnp.float32),
            pltpu.SemaphoreType.DMA,
            pltpu.SemaphoreType.DMA,
            pltpu.SemaphoreType.DMA,
            pltpu.SemaphoreType.DMA,
        ],
    )
    def sc_kernel(x_hbm, row_hbm, col_hbm, t_out, cnt_out,
                  rraw, mcol, rowbuf, rowbuf_b, zdeg, ones,
                  acc, deg, sem, sem_b, sem_sa, sem_sb):
        cid = lax.axis_index("c")
        sid = lax.axis_index("s")
        wid = cid * NS + sid
        ebase = wid * EPW
        dummy = N + sid

        zv = jnp.zeros((16,), jnp.float32)

        pltpu.sync_copy(col_hbm.at[pl.ds(ebase, EPW)], rraw.at[pl.ds(0, EPW)])

        def col_chunk(j, _):
            def col_vec(l, _):
                mcol[j, pl.ds(l * 16, 16)] = rraw[pl.ds(j * CHUNK + l * 16, 16)]
                return 0
            return lax.fori_loop(0, CHUNK // 16, col_vec, 0)
        lax.fori_loop(0, NREAL, col_chunk, 0)

        pltpu.sync_copy(row_hbm.at[pl.ds(ebase, EPW)], rraw.at[pl.ds(0, EPW)])

        dvec = jnp.full((16,), dummy, jnp.int32)

        def pad_tail(i, _):
            rraw[pl.ds(EPW + i * 16, 16)] = jnp.zeros((16,), jnp.int32)
            mcol[NREAL, pl.ds(i * 16, 16)] = dvec
            return 0
        lax.fori_loop(0, (EPAD - EPW) // 16, pad_tail, 0)

        def mask_chunk(j, _):
            def mask_vec(l, _):
                rv = rraw[pl.ds(j * CHUNK + l * 16, 16)]
                cv = mcol[j, pl.ds(l * 16, 16)]
                mcol[j, pl.ds(l * 16, 16)] = jnp.where(rv != cv, cv, dvec)
                return 0
            return lax.fori_loop(0, CHUNK // 16, mask_vec, 0)
        lax.fori_loop(0, NREAL, mask_chunk, 0)

        def zero_rowbuf(r, _):
            def zcol(c, _):
                rowbuf[r, pl.ds(c * 16, 16)] = zv
                return 0
            return lax.fori_loop(0, D // 16, zcol, 0)
        lax.fori_loop(0, CHUNK, zero_rowbuf, 0)

        def zdeg_fill(i, _):
            zdeg[pl.ds(i * 16, 16)] = zv
            return 0
        lax.fori_loop(0, RPT // 16, zdeg_fill, 0)

        def ones_fill(i, _):
            ones[pl.ds(i * 16, 16)] = jnp.ones((16,), jnp.float32)
            return 0
        lax.fori_loop(0, CHUNK // 16, ones_fill, 0)

        def zero_acc(t, _):
            pltpu.sync_copy(rowbuf, acc.at[pl.ds(sid * RPT + t * CHUNK, CHUNK)])
            return 0
        lax.fori_loop(0, RPT // CHUNK, zero_acc, 0)
        pltpu.sync_copy(zdeg, deg.at[pl.ds(sid * RPT, RPT)])

        plsc.subcore_barrier()

        HALF = CHUNK // 2

        def gather_start(j, buf, s):
            pltpu.async_copy(
                x_hbm.at[rraw.at[pl.ds(j * CHUNK, HALF)]],
                buf.at[pl.ds(0, HALF)], s)
            pltpu.async_copy(
                x_hbm.at[rraw.at[pl.ds(j * CHUNK + HALF, HALF)]],
                buf.at[pl.ds(HALF, HALF)], s)

        def gather_wait(j, buf, s):
            pltpu.make_async_copy(
                x_hbm.at[rraw.at[pl.ds(j * CHUNK, HALF)]],
                buf.at[pl.ds(0, HALF)], s).wait()
            pltpu.make_async_copy(
                x_hbm.at[rraw.at[pl.ds(j * CHUNK + HALF, HALF)]],
                buf.at[pl.ds(HALF, HALF)], s).wait()

        gather_start(0, rowbuf, sem)
        gather_start(1, rowbuf_b, sem_b)

        def edge_pair(t, _):
            j = t * 2
            gather_wait(j, rowbuf, sem)
            sca = pltpu.async_copy(rowbuf, acc.at[mcol.at[j]], sem_sa,
                                   add=True)
            pltpu.sync_copy(ones, deg.at[mcol.at[j]], add=True)

            gather_wait(j + 1, rowbuf_b, sem_b)
            scb = pltpu.async_copy(rowbuf_b, acc.at[mcol.at[j + 1]], sem_sb,
                                   add=True)
            pltpu.sync_copy(ones, deg.at[mcol.at[j + 1]], add=True)

            sca.wait()

            @pl.when(j + 2 < NCHUNK)
            def _():
                gather_start(j + 2, rowbuf, sem)

            scb.wait()

            @pl.when(j + 3 < NCHUNK)
            def _():
                gather_start(j + 3, rowbuf_b, sem_b)

            return 0
        lax.fori_loop(0, NCHUNK // 2, edge_pair, 0)

        plsc.subcore_barrier()

        rbase = sid * RPT
        pltpu.sync_copy(acc.at[pl.ds(rbase, RPT)],
                        t_out.at[pl.ds(cid * N_PAD + rbase, RPT)])
        pltpu.sync_copy(deg.at[pl.ds(rbase, RPT)],
                        cnt_out.at[pl.ds(cid * N_PAD + rbase, RPT)])

    return sc_kernel(x, row, col)


def _tc_combine(t0, t1, c0, c1, x, wout_t, wroot_t, b2d):
    RB = 400
    grid = (N // RB,)

    def tc_kernel(t0_ref, t1_ref, c0_ref, c1_ref, x_ref, wo_ref, wr_ref,
                  b_ref, o_ref):
        cnt = c0_ref[...] + c1_ref[...]
        inv = 1.0 / (cnt + 1.0)
        xb = x_ref[...]
        agg = (t0_ref[...] + t1_ref[...] + (1.0 + DIAG_LAMBDA) * xb) * inv
        acc = jnp.dot(agg, wo_ref[...], preferred_element_type=jnp.float32)
        acc += jnp.dot(xb, wr_ref[...], preferred_element_type=jnp.float32)
        o_ref[...] = jnp.maximum(acc + b_ref[...], 0.0)

    row_spec = pl.BlockSpec((RB, D), lambda i: (i, 0))
    return pl.pallas_call(
        tc_kernel,
        grid=grid,
        in_specs=[
            row_spec,
            row_spec,
            pl.BlockSpec((RB, 1), lambda i: (i, 0)),
            pl.BlockSpec((RB, 1), lambda i: (i, 0)),
            row_spec,
            pl.BlockSpec((D, D), lambda i: (0, 0)),
            pl.BlockSpec((D, D), lambda i: (0, 0)),
            pl.BlockSpec((1, D), lambda i: (0, 0)),
        ],
        out_specs=row_spec,
        out_shape=jax.ShapeDtypeStruct((N, D), jnp.float32),
    )(t0, t1, c0, c1, x, wout_t, wroot_t, b2d)


def kernel(x, x_0, edge_index, W_out, b_out, W_root):
    del x_0
    row = edge_index[0]
    col = edge_index[1]
    t_parts, cnt_parts = _sc_aggregate(x, row, col)
    t0 = t_parts[:N]
    t1 = t_parts[N_PAD:N_PAD + N]
    c0 = cnt_parts[:N].reshape(N, 1)
    c1 = cnt_parts[N_PAD:N_PAD + N].reshape(N, 1)
    return _tc_combine(t0, t1, c0, c1, x, W_out.T, W_root.T,
                       b_out.reshape(1, D))


# speedup vs baseline: 1.1378x; 1.1378x over previous
import functools

import jax
import jax.numpy as jnp
from jax import lax
from jax.experimental import pallas as pl
from jax.experimental.pallas import tpu as pltpu
from jax.experimental.pallas import tpu_sc as plsc

N = 10000
D = 128
E = 320000
DIAG_LAMBDA = 0.2

NC = 2
NS = 16
NW = NC * NS
EPW = E // NW
CHUNK = 96
NFULL = EPW // CHUNK
NCHUNK = 105
EPAD = NCHUNK * CHUNK
VPC = CHUNK // 16
ZD = 128
N_PAD = 10112
RPT = N_PAD // NS
NDB = N_PAD // ZD


def _sc_aggregate(x, row, col):
    mesh = plsc.VectorSubcoreMesh(core_axis_name="c", subcore_axis_name="s")

    @functools.partial(
        pl.kernel,
        out_type=(
            jax.ShapeDtypeStruct((NC * N_PAD, D), jnp.float32),
            jax.ShapeDtypeStruct((NC * N_PAD,), jnp.float32),
        ),
        mesh=mesh,
        scratch_types=[
            pltpu.VMEM((EPAD,), jnp.int32),
            pltpu.VMEM((NCHUNK, CHUNK), jnp.int32),
            pltpu.VMEM((CHUNK, D), jnp.float32),
            pltpu.VMEM((CHUNK, D), jnp.float32),
            pltpu.VMEM((ZD,), jnp.float32),
            pltpu.VMEM((CHUNK,), jnp.float32),
            pltpu.VMEM_SHARED((N_PAD, D), jnp.float32),
            pltpu.VMEM_SHARED((N_PAD,), jnp.float32),
            pltpu.SemaphoreType.DMA,
            pltpu.SemaphoreType.DMA,
        ],
    )
    def sc_kernel(x_hbm, row_hbm, col_hbm, t_out, cnt_out,
                  rraw, mcol, rowbuf, rowbuf_b, zdeg, ones,
                  acc, deg, sem, sem_b):
        cid = lax.axis_index("c")
        sid = lax.axis_index("s")
        wid = cid * NS + sid
        ebase = wid * EPW
        dummy = N + sid

        zv = jnp.zeros((16,), jnp.float32)

        pltpu.sync_copy(col_hbm.at[pl.ds(ebase, EPW)], rraw.at[pl.ds(0, EPW)])

        def col_chunk(j, _):
            def col_vec(l, _):
                mcol[j, pl.ds(l * 16, 16)] = rraw[pl.ds(j * CHUNK + l * 16, 16)]
                return 0
            return lax.fori_loop(0, VPC, col_vec, 0)
        lax.fori_loop(0, NFULL, col_chunk, 0)
        mcol[NFULL, pl.ds(0, 16)] = rraw[pl.ds(NFULL * CHUNK, 16)]

        pltpu.sync_copy(row_hbm.at[pl.ds(ebase, EPW)], rraw.at[pl.ds(0, EPW)])

        dvec = jnp.full((16,), dummy, jnp.int32)

        for i in range((EPAD - EPW) // 16):
            e = EPW + i * 16
            rraw[pl.ds(e, 16)] = jnp.zeros((16,), jnp.int32)
            mcol[e // CHUNK, pl.ds(e % CHUNK, 16)] = dvec

        def mask_chunk(j, _):
            def mask_vec(l, _):
                rv = rraw[pl.ds(j * CHUNK + l * 16, 16)]
                cv = mcol[j, pl.ds(l * 16, 16)]
                mcol[j, pl.ds(l * 16, 16)] = jnp.where(rv != cv, cv, dvec)
                return 0
            return lax.fori_loop(0, VPC, mask_vec, 0)
        lax.fori_loop(0, NFULL, mask_chunk, 0)
        rv_t = rraw[pl.ds(NFULL * CHUNK, 16)]
        cv_t = mcol[NFULL, pl.ds(0, 16)]
        mcol[NFULL, pl.ds(0, 16)] = jnp.where(rv_t != cv_t, cv_t, dvec)

        def zero_rowbuf(r, _):
            def zcol(c, _):
                rowbuf[r, pl.ds(c * 16, 16)] = zv
                return 0
            return lax.fori_loop(0, D // 16, zcol, 0)
        lax.fori_loop(0, CHUNK, zero_rowbuf, 0)

        def zdeg_fill(i, _):
            zdeg[pl.ds(i * 16, 16)] = zv
            return 0
        lax.fori_loop(0, ZD // 16, zdeg_fill, 0)

        def ones_fill(i, _):
            ones[pl.ds(i * 16, 16)] = jnp.ones((16,), jnp.float32)
            return 0
        lax.fori_loop(0, CHUNK // 16, ones_fill, 0)

        def zero_acc(t, _):
            pltpu.sync_copy(rowbuf, acc.at[pl.ds(sid * RPT + t * CHUNK, CHUNK)])
            return 0
        lax.fori_loop(0, RPT // CHUNK, zero_acc, 0)
        REM = RPT - (RPT // CHUNK) * CHUNK
        pltpu.sync_copy(rowbuf.at[pl.ds(0, REM)],
                        acc.at[pl.ds(sid * RPT + RPT - REM, REM)])
        for k in range(NDB):
            @pl.when(sid == k % NS)
            def _():
                pltpu.sync_copy(zdeg, deg.at[pl.ds(k * ZD, ZD)])

        plsc.subcore_barrier()

        HALF = CHUNK // 2

        def gather_start(j, buf, s):
            pltpu.async_copy(
                x_hbm.at[rraw.at[pl.ds(j * CHUNK, HALF)]],
                buf.at[pl.ds(0, HALF)], s)
            pltpu.async_copy(
                x_hbm.at[rraw.at[pl.ds(j * CHUNK + HALF, HALF)]],
                buf.at[pl.ds(HALF, HALF)], s)

        def gather_wait(j, buf, s):
            pltpu.make_async_copy(
                x_hbm.at[rraw.at[pl.ds(j * CHUNK, HALF)]],
                buf.at[pl.ds(0, HALF)], s).wait()
            pltpu.make_async_copy(
                x_hbm.at[rraw.at[pl.ds(j * CHUNK + HALF, HALF)]],
                buf.at[pl.ds(HALF, HALF)], s).wait()

        gather_start(0, rowbuf, sem)
        gather_start(1, rowbuf_b, sem_b)

        def edge_pair(t, _):
            g = t * 2
            for b, (buf, s) in enumerate(((rowbuf, sem), (rowbuf_b, sem_b))):
                j = g + b
                gather_wait(j, buf, s)
                pltpu.sync_copy(buf, acc.at[mcol.at[j]], add=True)

                @pl.when(j + 2 < NCHUNK)
                def _():
                    gather_start(j + 2, buf, s)

                pltpu.sync_copy(ones, deg.at[mcol.at[j]], add=True)
            return 0
        lax.fori_loop(0, NCHUNK // 2, edge_pair, 0)
        gather_wait(NCHUNK - 1, rowbuf, sem)
        pltpu.sync_copy(rowbuf, acc.at[mcol.at[NCHUNK - 1]], add=True)
        pltpu.sync_copy(ones, deg.at[mcol.at[NCHUNK - 1]], add=True)

        plsc.subcore_barrier()

        rbase = sid * RPT
        pltpu.sync_copy(acc.at[pl.ds(rbase, RPT)],
                        t_out.at[pl.ds(cid * N_PAD + rbase, RPT)])

        for k in range(NDB):
            @pl.when(sid == k % NS)
            def _():
                pltpu.sync_copy(deg.at[pl.ds(k * ZD, ZD)],
                                cnt_out.at[pl.ds(cid * N_PAD + k * ZD, ZD)])

    return sc_kernel(x, row, col)


def _tc_combine(t0, t1, c0, c1, x, wout_t, wroot_t, b2d):
    RB = 400
    grid = (N // RB,)

    def tc_kernel(t0_ref, t1_ref, c0_ref, c1_ref, x_ref, wo_ref, wr_ref,
                  b_ref, o_ref):
        cnt = c0_ref[...] + c1_ref[...]
        inv = 1.0 / (cnt + 1.0)
        xb = x_ref[...]
        agg = (t0_ref[...] + t1_ref[...] + (1.0 + DIAG_LAMBDA) * xb) * inv
        acc = jnp.dot(agg, wo_ref[...], preferred_element_type=jnp.float32)
        acc += jnp.dot(xb, wr_ref[...], preferred_element_type=jnp.float32)
        o_ref[...] = jnp.maximum(acc + b_ref[...], 0.0)

    row_spec = pl.BlockSpec((RB, D), lambda i: (i, 0))
    return pl.pallas_call(
        tc_kernel,
        grid=grid,
        in_specs=[
            row_spec,
            row_spec,
            pl.BlockSpec((RB, 1), lambda i: (i, 0)),
            pl.BlockSpec((RB, 1), lambda i: (i, 0)),
            row_spec,
            pl.BlockSpec((D, D), lambda i: (0, 0)),
            pl.BlockSpec((D, D), lambda i: (0, 0)),
            pl.BlockSpec((1, D), lambda i: (0, 0)),
        ],
        out_specs=row_spec,
        out_shape=jax.ShapeDtypeStruct((N, D), jnp.float32),
    )(t0, t1, c0, c1, x, wout_t, wroot_t, b2d)


def kernel(x, x_0, edge_index, W_out, b_out, W_root):
    del x_0
    row = edge_index[0]
    col = edge_index[1]
    t_parts, cnt_parts = _sc_aggregate(x, row, col)
    t0 = t_parts[:N]
    t1 = t_parts[N_PAD:N_PAD + N]
    c0 = cnt_parts[:N].reshape(N, 1)
    c1 = cnt_parts[N_PAD:N_PAD + N].reshape(N, 1)
    return _tc_combine(t0, t1, c0, c1, x, W_out.T, W_root.T,
                       b_out.reshape(1, D))
